# Initial kernel scaffold; baseline (speedup 1.0000x reference)
import functools

import jax
import jax.numpy as jnp
from jax import lax
from jax.experimental import pallas as pl
from jax.experimental.pallas import tpu as pltpu
from jax.experimental.pallas import tpu_sc as plsc

D = 51
CHUNK = 128


def _make_gather(B: int):
    info = plsc.get_sparse_core_info()
    NC, NS = info.num_cores, info.num_subcores
    NW = NC * NS
    assert B % (NW * CHUNK) == 0
    per_w = B // NW
    n_chunks = per_w // CHUNK
    mesh = plsc.VectorSubcoreMesh(core_axis_name="c", subcore_axis_name="s")

    @functools.partial(
        pl.kernel,
        mesh=mesh,
        out_type=jax.ShapeDtypeStruct((B, D), jnp.float32),
        scratch_types=[
            pltpu.VMEM((CHUNK,), jnp.int32),
            pltpu.VMEM((CHUNK, D), jnp.float32),
            pltpu.SemaphoreType.DMA,
        ],
    )
    def gather_kernel(table_hbm, idx_hbm, out_hbm, idx_v, rows_v, sem):
        wid = lax.axis_index("s") * NC + lax.axis_index("c")
        base = wid * per_w

        def body(c, carry):
            start = base + c * CHUNK
            pltpu.sync_copy(idx_hbm.at[pl.ds(start, CHUNK)], idx_v)
            pltpu.async_copy(table_hbm.at[idx_v], rows_v, sem).wait()
            pltpu.sync_copy(rows_v, out_hbm.at[pl.ds(start, CHUNK), :])
            return carry

        lax.fori_loop(0, n_chunks, body, 0)

    return gather_kernel


def kernel(colors, table, onehot_matrix, prop_matrix):
    fused = jnp.concatenate([table, onehot_matrix, prop_matrix], axis=1)
    B = colors.size
    idx = colors.reshape(B).astype(jnp.int32)
    out = _make_gather(B)(fused, idx)
    return out.reshape(colors.shape + (D,))


# speedup vs baseline: 12.4887x; 12.4887x over previous
import functools

import jax
import jax.numpy as jnp
from jax import lax
from jax.experimental import pallas as pl
from jax.experimental.pallas import tpu as pltpu
from jax.experimental.pallas import tpu_sc as plsc

D = 51
TPAD = 512
CH = 576


def _make_gather(B: int):
    info = plsc.get_sparse_core_info()
    NC, NS, L = info.num_cores, info.num_subcores, info.num_lanes
    NW = NC * NS
    assert B % (NW * CH) == 0 and CH % L == 0
    per_w = B // NW
    n_chunks = per_w // CH
    n_groups = CH // L
    mesh = plsc.VectorSubcoreMesh(core_axis_name="c", subcore_axis_name="s")

    @functools.partial(
        pl.kernel,
        mesh=mesh,
        compiler_params=pltpu.CompilerParams(
            use_tc_tiling_on_sc=False, needs_layout_passes=False),
        out_type=jax.ShapeDtypeStruct((B * D,), jnp.float32),
        scratch_types=[
            pltpu.VMEM((TPAD,), jnp.float32),
            pltpu.VMEM((CH,), jnp.int32),
            pltpu.VMEM((CH * D,), jnp.float32),
            pltpu.SemaphoreType.DMA,
        ],
    )
    def gather_kernel(table_hbm, idx_hbm, out_hbm, table_v, idx_v, out_v, sem):
        wid = lax.axis_index("s") * NC + lax.axis_index("c")
        base = wid * per_w
        pltpu.sync_copy(table_hbm, table_v)
        lane_rows = lax.iota(jnp.int32, L) * D

        def chunk_body(c, carry):
            start = base + c * CH
            pltpu.sync_copy(idx_hbm.at[pl.ds(start, CH)], idx_v)

            def group_body(g, carry2):
                colors_v = idx_v[pl.ds(g * L, L)]
                src0 = colors_v * D
                dst0 = lane_rows + g * (L * D)
                for j in range(D):
                    v = plsc.load_gather(table_v, [src0 + j])
                    plsc.store_scatter(out_v, [dst0 + j], v)
                return carry2

            lax.fori_loop(0, n_groups, group_body, 0)
            pltpu.sync_copy(out_v, out_hbm.at[pl.ds(start * D, CH * D)])
            return carry

        lax.fori_loop(0, n_chunks, chunk_body, 0)

    return gather_kernel


def kernel(colors, table, onehot_matrix, prop_matrix):
    fused = jnp.concatenate([table, onehot_matrix, prop_matrix], axis=1)
    flat = jnp.concatenate(
        [fused.reshape(-1), jnp.zeros((TPAD - fused.size,), jnp.float32)])
    B = colors.size
    idx = colors.reshape(B).astype(jnp.int32)
    out = _make_gather(B)(flat, idx)
    return out.reshape(colors.shape + (D,))


# speedup vs baseline: 17.1770x; 1.3754x over previous
import functools

import jax
import jax.numpy as jnp
from jax import lax
from jax.experimental import pallas as pl
from jax.experimental.pallas import tpu as pltpu
from jax.experimental.pallas import tpu_sc as plsc

D = 51
TPAD = 512
CH = 576


def _make_gather(B: int):
    info = plsc.get_sparse_core_info()
    NC, NS, L = info.num_cores, info.num_subcores, info.num_lanes
    NW = NC * NS
    assert B % (NW * CH * 2) == 0 and CH % L == 0
    per_w = B // NW
    n_chunks = per_w // CH
    n_groups = CH // L
    mesh = plsc.VectorSubcoreMesh(core_axis_name="c", subcore_axis_name="s")

    @functools.partial(
        pl.kernel,
        mesh=mesh,
        compiler_params=pltpu.CompilerParams(
            use_tc_tiling_on_sc=False, needs_layout_passes=False),
        out_type=jax.ShapeDtypeStruct((B * D,), jnp.float32),
        scratch_types=[
            pltpu.VMEM((TPAD,), jnp.float32),
            pltpu.VMEM((CH,), jnp.int32),
            pltpu.VMEM((CH,), jnp.int32),
            pltpu.VMEM((CH * D,), jnp.float32),
            pltpu.VMEM((CH * D,), jnp.float32),
            pltpu.SemaphoreType.DMA,
            pltpu.SemaphoreType.DMA,
            pltpu.SemaphoreType.DMA,
            pltpu.SemaphoreType.DMA,
        ],
    )
    def gather_kernel(table_hbm, idx_hbm, out_hbm, table_v,
                      idx_v0, idx_v1, out_v0, out_v1,
                      sem_i0, sem_i1, sem_o0, sem_o1):
        wid = lax.axis_index("s") * NC + lax.axis_index("c")
        base = wid * per_w
        pltpu.sync_copy(table_hbm, table_v)
        lane_rows = lax.iota(jnp.int32, L) * D

        idx_bufs = (idx_v0, idx_v1)
        out_bufs = (out_v0, out_v1)
        sem_i = (sem_i0, sem_i1)
        sem_o = (sem_o0, sem_o1)

        def idx_copy(c, p):
            start = base + c * CH
            return pltpu.make_async_copy(
                idx_hbm.at[pl.ds(start, CH)], idx_bufs[p], sem_i[p])

        def out_copy(c, p):
            start = base + c * CH
            return pltpu.make_async_copy(
                out_bufs[p], out_hbm.at[pl.ds(start * D, CH * D)], sem_o[p])

        idx_copy(0, 0).start()

        def super_body(s, carry):
            for p in range(2):
                c = 2 * s + p
                idx_copy(c, p).wait()

                @pl.when(c + 1 < n_chunks)
                def _():
                    idx_copy(c + 1, 1 - p).start()

                @pl.when(c >= 2)
                def _():
                    out_copy(c - 2, p).wait()

                idx_v = idx_bufs[p]
                out_v = out_bufs[p]

                @plsc.parallel_loop(0, n_groups, unroll=4)
                def group_body(g):
                    colors_v = idx_v[pl.ds(g * L, L)]
                    src0 = colors_v * D
                    dst0 = lane_rows + g * (L * D)
                    for j in range(D):
                        v = plsc.load_gather(table_v, [src0 + j])
                        plsc.store_scatter(out_v, [dst0 + j], v)

                out_copy(c, p).start()
            return carry

        lax.fori_loop(0, n_chunks // 2, super_body, 0)
        out_copy(n_chunks - 2, 0).wait()
        out_copy(n_chunks - 1, 1).wait()

    return gather_kernel


def kernel(colors, table, onehot_matrix, prop_matrix):
    fused = jnp.concatenate([table, onehot_matrix, prop_matrix], axis=1)
    flat = jnp.concatenate(
        [fused.reshape(-1), jnp.zeros((TPAD - fused.size,), jnp.float32)])
    B = colors.size
    idx = colors.reshape(B).astype(jnp.int32)
    out = _make_gather(B)(flat, idx)
    return out.reshape(colors.shape + (D,))
